# baseline (speedup 1.0000x reference)
import jax
import jax.numpy as jnp
from jax.experimental import pallas as pl
from jax.experimental.pallas import tpu as pltpu

_IN, _H, _OUT = 4, 8, 3
_N_BLOCKS = 4


def _mlp_t_body(xt_ref, m1_ref, m2_ref, m3_ref, m4t_ref, b_ref, o_ref):
    c1 = jnp.transpose(m1_ref[0:_IN, 0:_H])
    c2 = jnp.transpose(m2_ref[0:_H, 0:_H])
    c3 = jnp.transpose(m3_ref[0:_H, 0:_H])
    c4 = m4t_ref[0:_OUT, 0:_H]
    bt = jnp.transpose(b_ref[0:4, 0:_H])

    x = xt_ref[...]

    f32 = jnp.float32
    h = jnp.dot(c1, x, preferred_element_type=f32) + bt[:, 0:1]
    h = jnp.maximum(h, 0.0)
    h = jnp.dot(c2, h, preferred_element_type=f32) + bt[:, 1:2]
    h = jnp.maximum(h, 0.0)
    h = jnp.dot(c3, h, preferred_element_type=f32) + bt[:, 2:3]
    h = jnp.maximum(h, 0.0)
    o_ref[...] = jnp.dot(c4, h, preferred_element_type=f32) + bt[0:_OUT, 3:4]


def kernel(x, m1, m2, m3, m4, biases):
    B = x.shape[0]
    xt = jnp.swapaxes(jnp.asarray(x, jnp.float32), 0, 1)

    lt = pl.cdiv(B, _N_BLOCKS)
    lt = ((lt + 127) // 128) * 128
    n_blocks = pl.cdiv(B, lt)
    b_pad = lt * n_blocks
    if b_pad != B:
        xt = jnp.pad(xt, ((0, 0), (0, b_pad - B)))

    out_t = pl.pallas_call(
        _mlp_t_body,
        out_shape=jax.ShapeDtypeStruct((_OUT, b_pad), jnp.float32),
        grid_spec=pl.GridSpec(
            grid=(n_blocks,),
            in_specs=[
                pl.BlockSpec((_IN, lt), lambda i: (0, i)),
                pl.BlockSpec(m1.shape, lambda i: (0, 0)),
                pl.BlockSpec(m2.shape, lambda i: (0, 0)),
                pl.BlockSpec(m3.shape, lambda i: (0, 0)),
                pl.BlockSpec((m4.shape[1], m4.shape[0]), lambda i: (0, 0)),
                pl.BlockSpec(biases.shape, lambda i: (0, 0)),
            ],
            out_specs=pl.BlockSpec((_OUT, lt), lambda i: (0, i)),
        ),
        compiler_params=pltpu.CompilerParams(
            dimension_semantics=("parallel",),
            vmem_limit_bytes=64 * 1024 * 1024,
        ),
    )(xt, jnp.asarray(m1, jnp.float32), jnp.asarray(m2, jnp.float32),
      jnp.asarray(m3, jnp.float32),
      jnp.swapaxes(jnp.asarray(m4, jnp.float32), 0, 1),
      jnp.asarray(biases, jnp.float32))

    out = jnp.swapaxes(out_t, 0, 1)
    return out[:B] if b_pad != B else out


# speedup vs baseline: 1.0299x; 1.0299x over previous
import jax
import jax.numpy as jnp
from jax.experimental import pallas as pl
from jax.experimental.pallas import tpu as pltpu

_IN, _H, _OUT = 4, 8, 3
_N_BLOCKS = 8


def _mlp_t_body(xt_ref, m1_ref, m2_ref, m3_ref, m4t_ref, b_ref, o_ref):
    c1 = jnp.transpose(m1_ref[0:_IN, 0:_H])
    c2 = jnp.transpose(m2_ref[0:_H, 0:_H])
    c3 = jnp.transpose(m3_ref[0:_H, 0:_H])
    c4 = m4t_ref[0:_OUT, 0:_H]
    bt = jnp.transpose(b_ref[0:4, 0:_H])

    x = xt_ref[...]

    f32 = jnp.float32
    h = jnp.dot(c1, x, preferred_element_type=f32) + bt[:, 0:1]
    h = jnp.maximum(h, 0.0)
    h = jnp.dot(c2, h, preferred_element_type=f32) + bt[:, 1:2]
    h = jnp.maximum(h, 0.0)
    h = jnp.dot(c3, h, preferred_element_type=f32) + bt[:, 2:3]
    h = jnp.maximum(h, 0.0)
    o_ref[...] = jnp.dot(c4, h, preferred_element_type=f32) + bt[0:_OUT, 3:4]


def kernel(x, m1, m2, m3, m4, biases):
    B = x.shape[0]
    xt = jnp.swapaxes(jnp.asarray(x, jnp.float32), 0, 1)

    lt = pl.cdiv(B, _N_BLOCKS)
    lt = ((lt + 127) // 128) * 128
    n_blocks = pl.cdiv(B, lt)
    b_pad = lt * n_blocks
    if b_pad != B:
        xt = jnp.pad(xt, ((0, 0), (0, b_pad - B)))

    out_t = pl.pallas_call(
        _mlp_t_body,
        out_shape=jax.ShapeDtypeStruct((_OUT, b_pad), jnp.float32),
        grid_spec=pl.GridSpec(
            grid=(n_blocks,),
            in_specs=[
                pl.BlockSpec((_IN, lt), lambda i: (0, i)),
                pl.BlockSpec(m1.shape, lambda i: (0, 0)),
                pl.BlockSpec(m2.shape, lambda i: (0, 0)),
                pl.BlockSpec(m3.shape, lambda i: (0, 0)),
                pl.BlockSpec((m4.shape[1], m4.shape[0]), lambda i: (0, 0)),
                pl.BlockSpec(biases.shape, lambda i: (0, 0)),
            ],
            out_specs=pl.BlockSpec((_OUT, lt), lambda i: (0, i)),
        ),
        compiler_params=pltpu.CompilerParams(
            dimension_semantics=("parallel",),
            vmem_limit_bytes=64 * 1024 * 1024,
        ),
    )(xt, jnp.asarray(m1, jnp.float32), jnp.asarray(m2, jnp.float32),
      jnp.asarray(m3, jnp.float32),
      jnp.swapaxes(jnp.asarray(m4, jnp.float32), 0, 1),
      jnp.asarray(biases, jnp.float32))

    out = jnp.swapaxes(out_t, 0, 1)
    return out[:B] if b_pad != B else out
